# baseline (speedup 1.0000x reference)
import functools

import jax
import jax.numpy as jnp
from jax import lax
from jax.experimental import pallas as pl
from jax.experimental.pallas import tpu as pltpu
from jax.experimental.pallas import tpu_sc as plsc

NUM_CORES = 2
NUM_SUBCORES = 16
NW = NUM_CORES * NUM_SUBCORES

CHUNK = 128
NBUF = 2
TBLK = 1024


def _pack_kernel(tT_ref, out_ref):
    t1 = tT_ref[...].T
    out_ref[:, 0:64] = t1[0:TBLK // 2]
    out_ref[:, 64:128] = t1[TBLK // 2:TBLK]


def _pack_table(tableT, n_rows):
    n_blocks = (n_rows + TBLK - 1) // TBLK
    return pl.pallas_call(
        _pack_kernel,
        grid=(n_blocks,),
        in_specs=[pl.BlockSpec((64, TBLK), lambda i: (0, i))],
        out_specs=pl.BlockSpec((TBLK // 2, 128), lambda i: (i, 0)),
        out_shape=jax.ShapeDtypeStruct((n_blocks * (TBLK // 2), 128),
                                       tableT.dtype),
    )(tableT)


def _gather_kernel(hist, d, idxT_hbm, tableH_hbm, out5_hbm, idx_v, idxh_v,
                   par_v, rows_v, rowsT_v, gsems, ssems):
    wid = lax.axis_index("s") * NUM_CORES + lax.axis_index("c")

    pltpu.sync_copy(idxT_hbm.at[:, pl.ds(wid * CHUNK, CHUNK)], idx_v)

    def prep_row(h, _):
        for bb in range(8):
            v = idx_v[h, pl.ds(bb * 16, 16)]
            idxh_v[h, pl.ds(bb * 16, 16)] = ((v >> 10) << 9) | (v & 511)
            par_v[h, pl.ds(bb * 16, 16)] = ((v >> 9) & 1) << 6
        return 0

    lax.fori_loop(0, hist, prep_row, 0)

    def start_gather(h, buf):
        pltpu.async_copy(
            tableH_hbm.at[idxh_v.at[h]],
            rows_v.at[buf],
            gsems.at[buf],
        )

    def wait_gather(h, buf):
        pltpu.make_async_copy(
            tableH_hbm.at[idxh_v.at[h]],
            rows_v.at[buf],
            gsems.at[buf],
        ).wait()

    def start_store(h, buf):
        pltpu.async_copy(
            rowsT_v.at[buf],
            out5_hbm.at[h, :, wid],
            ssems.at[buf],
        )

    def wait_store(h, buf):
        pltpu.make_async_copy(
            rowsT_v.at[buf],
            out5_hbm.at[h, :, wid],
            ssems.at[buf],
        ).wait()

    lane = lax.iota(jnp.int32, 16)
    row_idx = [lane + (16 * bb) for bb in range(8)]

    def transpose_chunk(h, buf):
        rows = rows_v.at[buf]

        def jt_body(jt, _):
            for r in range(8):
                dd = jt * 8 + r
                for bb in range(8):
                    par = par_v[h, pl.ds(bb * 16, 16)]
                    v = plsc.load_gather(rows, [row_idx[bb], par + dd])
                    rowsT_v[buf, jt, r, pl.ds(bb * 16, 16)] = v
            return 0

        lax.fori_loop(0, 8, jt_body, 0)

    start_gather(0, 0)
    start_gather(1, 1)

    def body(g, _):
        for buf in range(NBUF):
            h = g * NBUF + buf
            wait_gather(h, buf)

            @pl.when(h >= NBUF)
            def _():
                wait_store(h - NBUF, buf)

            transpose_chunk(h, buf)
            start_store(h, buf)

            @pl.when(h + NBUF < hist)
            def _():
                start_gather(h + NBUF, buf)

        return 0

    lax.fori_loop(0, hist // NBUF, body, 0)

    for buf in range(NBUF):
        wait_store(hist - NBUF + buf, buf)


def kernel(item_ids, table):
    batch, hist = item_ids.shape
    n_rows, d = table.shape
    idxT = item_ids.T
    tableH = _pack_table(table.T, n_rows)

    mesh = plsc.VectorSubcoreMesh(
        core_axis_name="c",
        subcore_axis_name="s",
        num_cores=NUM_CORES,
        num_subcores=NUM_SUBCORES,
    )

    grid_kernel = pl.kernel(
        functools.partial(_gather_kernel, hist, d),
        out_type=jax.ShapeDtypeStruct((hist, 8, batch // CHUNK, 8, CHUNK),
                                      table.dtype),
        mesh=mesh,
        scratch_types=[
            pltpu.VMEM((hist, CHUNK), jnp.int32),
            pltpu.VMEM((hist, CHUNK), jnp.int32),
            pltpu.VMEM((hist, CHUNK), jnp.int32),
            pltpu.VMEM((NBUF, CHUNK, 128), table.dtype),
            pltpu.VMEM((NBUF, 8, 8, CHUNK), table.dtype),
            pltpu.SemaphoreType.DMA((NBUF,)),
            pltpu.SemaphoreType.DMA((NBUF,)),
        ],
        compiler_params=pltpu.CompilerParams(
            use_tc_tiling_on_sc=False, needs_layout_passes=False
        ),
    )
    out5 = grid_kernel(idxT, tableH)
    return out5.transpose(2, 4, 0, 1, 3).reshape(batch, hist, d)


# speedup vs baseline: 2.2772x; 2.2772x over previous
import functools

import jax
import jax.numpy as jnp
from jax import lax
from jax.experimental import pallas as pl
from jax.experimental.pallas import tpu as pltpu
from jax.experimental.pallas import tpu_sc as plsc

NUM_CORES = 2
NUM_SUBCORES = 16
NW = NUM_CORES * NUM_SUBCORES

CHUNK = 128
NBUF = 2
TBLK = 1024


def _pack_kernel(tT_ref, out_ref):
    t1 = tT_ref[...].T
    out_ref[:, 0:64] = t1[0:TBLK // 2]
    out_ref[:, 64:128] = t1[TBLK // 2:TBLK]


def _pack_table(tableT, n_rows):
    n_blocks = (n_rows + TBLK - 1) // TBLK
    return pl.pallas_call(
        _pack_kernel,
        grid=(n_blocks,),
        in_specs=[pl.BlockSpec((64, TBLK), lambda i: (0, i))],
        out_specs=pl.BlockSpec((TBLK // 2, 128), lambda i: (i, 0)),
        out_shape=jax.ShapeDtypeStruct((n_blocks * (TBLK // 2), 128),
                                       tableT.dtype),
    )(tableT)


def _gather_kernel(hist, d, idxT_hbm, tableH_hbm, out5_hbm, idxh_v,
                   par_v, rows_v, rowsT_v, gsems, ssems):
    wid = lax.axis_index("s") * NUM_CORES + lax.axis_index("c")

    pltpu.sync_copy(idxT_hbm.at[:, pl.ds(wid * CHUNK, CHUNK)], idxh_v)

    def prep_row(h, _):
        for bb in range(8):
            v = idxh_v[h, pl.ds(bb * 16, 16)]
            par_v[h, pl.ds(bb * 16, 16)] = ((v >> 9) & 1) << 6
            idxh_v[h, pl.ds(bb * 16, 16)] = ((v >> 10) << 9) | (v & 511)
        return 0

    lax.fori_loop(0, hist, prep_row, 0)

    def start_gather(h, buf):
        pltpu.async_copy(
            tableH_hbm.at[idxh_v.at[h]],
            rows_v.at[buf],
            gsems.at[buf],
        )

    def wait_gather(h, buf):
        pltpu.make_async_copy(
            tableH_hbm.at[idxh_v.at[h]],
            rows_v.at[buf],
            gsems.at[buf],
        ).wait()

    def start_store(h, buf):
        for jt in range(8):
            pltpu.async_copy(
                rowsT_v.at[buf, pl.ds(jt * 8, 8)],
                out5_hbm.at[h, jt, wid],
                ssems.at[buf],
            )

    def wait_store(h, buf):
        for jt in range(8):
            pltpu.make_async_copy(
                rowsT_v.at[buf, pl.ds(jt * 8, 8)],
                out5_hbm.at[h, jt, wid],
                ssems.at[buf],
            ).wait()

    lane = lax.iota(jnp.int32, 16)

    def transpose_chunk(h, buf):
        rows = rows_v.at[buf]
        rowsT = rowsT_v.at[buf]
        c_vecs = [lane + (16 * bb) for bb in range(8)]
        pard0 = [[par_v[h, pl.ds(bb * 16, 16)] + d0 for d0 in range(0, 64, 16)]
                 for bb in range(8)]

        def k_body(k, _):
            dg = (lane + k) & 15
            for bb in range(8):
                for di, d0 in enumerate(range(0, 64, 16)):
                    v = plsc.load_gather(rows, [c_vecs[bb], pard0[bb][di] + dg])
                    plsc.store_scatter(rowsT, [dg + d0, c_vecs[bb]], v)
            return 0

        lax.fori_loop(0, 16, k_body, 0)

    start_gather(0, 0)
    start_gather(1, 1)

    def body(g, _):
        for buf in range(NBUF):
            h = g * NBUF + buf
            wait_gather(h, buf)

            @pl.when(h >= NBUF)
            def _():
                wait_store(h - NBUF, buf)

            transpose_chunk(h, buf)
            start_store(h, buf)

            @pl.when(h + NBUF < hist)
            def _():
                start_gather(h + NBUF, buf)

        return 0

    lax.fori_loop(0, hist // NBUF, body, 0)

    for buf in range(NBUF):
        wait_store(hist - NBUF + buf, buf)


def kernel(item_ids, table):
    batch, hist = item_ids.shape
    n_rows, d = table.shape
    idxT = item_ids.T
    tableH = _pack_table(table.T, n_rows)

    mesh = plsc.VectorSubcoreMesh(
        core_axis_name="c",
        subcore_axis_name="s",
        num_cores=NUM_CORES,
        num_subcores=NUM_SUBCORES,
    )

    grid_kernel = pl.kernel(
        functools.partial(_gather_kernel, hist, d),
        out_type=jax.ShapeDtypeStruct((hist, 8, batch // CHUNK, 8, CHUNK),
                                      table.dtype),
        mesh=mesh,
        scratch_types=[
            pltpu.VMEM((hist, CHUNK), jnp.int32),
            pltpu.VMEM((hist, CHUNK), jnp.int32),
            pltpu.VMEM((NBUF, CHUNK, 128), table.dtype),
            pltpu.VMEM((NBUF, 64, CHUNK), table.dtype),
            pltpu.SemaphoreType.DMA((NBUF,)),
            pltpu.SemaphoreType.DMA((NBUF,)),
        ],
        compiler_params=pltpu.CompilerParams(
            use_tc_tiling_on_sc=False, needs_layout_passes=False
        ),
    )
    out5 = grid_kernel(idxT, tableH)
    return out5.transpose(2, 4, 0, 1, 3).reshape(batch, hist, d)


# speedup vs baseline: 2.6382x; 1.1585x over previous
import functools

import jax
import jax.numpy as jnp
from jax import lax
from jax.experimental import pallas as pl
from jax.experimental.pallas import tpu as pltpu
from jax.experimental.pallas import tpu_sc as plsc

NUM_CORES = 2
NUM_SUBCORES = 16
NW = NUM_CORES * NUM_SUBCORES

CHUNK = 128
NBUF = 2
TBLK = 1024


def _pack_kernel(tT_ref, out_ref):
    blk = tT_ref[...]
    eye = (lax.broadcasted_iota(jnp.int32, (64, 64), 0)
           == lax.broadcasted_iota(jnp.int32, (64, 64), 1)).astype(blk.dtype)
    t1 = lax.dot_general(blk, eye, (((0,), (0,)), ((), ())),
                         preferred_element_type=jnp.float32)
    out_ref[:, 0:64] = t1[0:TBLK // 2]
    out_ref[:, 64:128] = t1[TBLK // 2:TBLK]


def _pack_table(tableT, n_rows):
    n_blocks = (n_rows + TBLK - 1) // TBLK
    return pl.pallas_call(
        _pack_kernel,
        grid=(n_blocks,),
        in_specs=[pl.BlockSpec((64, TBLK), lambda i: (0, i))],
        out_specs=pl.BlockSpec((TBLK // 2, 128), lambda i: (i, 0)),
        out_shape=jax.ShapeDtypeStruct((n_blocks * (TBLK // 2), 128),
                                       tableT.dtype),
    )(tableT)


def _gather_kernel(hist, d, idxT_hbm, tableH_hbm, out5_hbm, idxh_v,
                   par_v, rows_v, rowsT_v, gsems, ssems):
    wid = lax.axis_index("s") * NUM_CORES + lax.axis_index("c")

    pltpu.sync_copy(idxT_hbm.at[:, pl.ds(wid * CHUNK, CHUNK)], idxh_v)

    def prep_row(h, _):
        for bb in range(8):
            v = idxh_v[h, pl.ds(bb * 16, 16)]
            par_v[h, pl.ds(bb * 16, 16)] = ((v >> 9) & 1) << 6
            idxh_v[h, pl.ds(bb * 16, 16)] = ((v >> 10) << 9) | (v & 511)
        return 0

    lax.fori_loop(0, hist, prep_row, 0)

    def start_gather(h, buf):
        pltpu.async_copy(
            tableH_hbm.at[idxh_v.at[h]],
            rows_v.at[buf],
            gsems.at[buf],
        )

    def wait_gather(h, buf):
        pltpu.make_async_copy(
            tableH_hbm.at[idxh_v.at[h]],
            rows_v.at[buf],
            gsems.at[buf],
        ).wait()

    def start_store(h, buf):
        for jt in range(8):
            pltpu.async_copy(
                rowsT_v.at[buf, pl.ds(jt * 8, 8)],
                out5_hbm.at[h, jt, wid],
                ssems.at[buf],
            )

    def wait_store(h, buf):
        for jt in range(8):
            pltpu.make_async_copy(
                rowsT_v.at[buf, pl.ds(jt * 8, 8)],
                out5_hbm.at[h, jt, wid],
                ssems.at[buf],
            ).wait()

    lane = lax.iota(jnp.int32, 16)

    def transpose_chunk(h, buf):
        rows = rows_v.at[buf]
        rowsT = rowsT_v.at[buf]
        c_vecs = [lane + (16 * bb) for bb in range(8)]
        pars = [par_v[h, pl.ds(bb * 16, 16)] for bb in range(8)]

        def k_body(k, _):
            dg = (lane + k) & 15
            dgd0 = [dg + d0 for d0 in range(0, 64, 16)]
            vals = [
                plsc.load_gather(rows, [c_vecs[bb], pars[bb] + dgd0[di]])
                for bb in range(8) for di in range(4)
            ]
            i = 0
            for bb in range(8):
                for di in range(4):
                    plsc.store_scatter(rowsT, [dgd0[di], c_vecs[bb]], vals[i])
                    i += 1
            return 0

        lax.fori_loop(0, 16, k_body, 0)

    start_gather(0, 0)
    start_gather(1, 1)

    def body(g, _):
        for buf in range(NBUF):
            h = g * NBUF + buf
            wait_gather(h, buf)

            @pl.when(h >= NBUF)
            def _():
                wait_store(h - NBUF, buf)

            transpose_chunk(h, buf)
            start_store(h, buf)

            @pl.when(h + NBUF < hist)
            def _():
                start_gather(h + NBUF, buf)

        return 0

    lax.fori_loop(0, hist // NBUF, body, 0)

    for buf in range(NBUF):
        wait_store(hist - NBUF + buf, buf)


def kernel(item_ids, table):
    batch, hist = item_ids.shape
    n_rows, d = table.shape
    idxT = item_ids.T
    tableH = _pack_table(table.T, n_rows)

    mesh = plsc.VectorSubcoreMesh(
        core_axis_name="c",
        subcore_axis_name="s",
        num_cores=NUM_CORES,
        num_subcores=NUM_SUBCORES,
    )

    grid_kernel = pl.kernel(
        functools.partial(_gather_kernel, hist, d),
        out_type=jax.ShapeDtypeStruct((hist, 8, batch // CHUNK, 8, CHUNK),
                                      table.dtype),
        mesh=mesh,
        scratch_types=[
            pltpu.VMEM((hist, CHUNK), jnp.int32),
            pltpu.VMEM((hist, CHUNK), jnp.int32),
            pltpu.VMEM((NBUF, CHUNK, 128), table.dtype),
            pltpu.VMEM((NBUF, 64, CHUNK), table.dtype),
            pltpu.SemaphoreType.DMA((NBUF,)),
            pltpu.SemaphoreType.DMA((NBUF,)),
        ],
        compiler_params=pltpu.CompilerParams(
            use_tc_tiling_on_sc=False, needs_layout_passes=False
        ),
    )
    out5 = grid_kernel(idxT, tableH)
    return out5.transpose(2, 4, 0, 1, 3).reshape(batch, hist, d)


# speedup vs baseline: 4.3213x; 1.6380x over previous
import functools

import jax
import jax.numpy as jnp
from jax import lax
from jax.experimental import pallas as pl
from jax.experimental.pallas import tpu as pltpu
from jax.experimental.pallas import tpu_sc as plsc

NUM_CORES = 2
NUM_SUBCORES = 16
NW = NUM_CORES * NUM_SUBCORES

CHUNK = 128
NBUF = 2
TBLK = 4096


def _pack_kernel(tT_ref, out_ref):
    blk = tT_ref[...]
    eye = (lax.broadcasted_iota(jnp.int32, (64, 64), 0)
           == lax.broadcasted_iota(jnp.int32, (64, 64), 1)).astype(blk.dtype)
    t1 = lax.dot_general(blk, eye, (((0,), (0,)), ((), ())),
                         preferred_element_type=jnp.float32)
    out_ref[:, 0:64] = t1[0:TBLK // 2]
    out_ref[:, 64:128] = t1[TBLK // 2:TBLK]


def _pack_table(tableT, n_rows):
    n_blocks = (n_rows + TBLK - 1) // TBLK
    return pl.pallas_call(
        _pack_kernel,
        grid=(n_blocks,),
        in_specs=[pl.BlockSpec((64, TBLK), lambda i: (0, i))],
        out_specs=pl.BlockSpec((TBLK // 2, 128), lambda i: (i, 0)),
        out_shape=jax.ShapeDtypeStruct((n_blocks * (TBLK // 2), 128),
                                       tableT.dtype),
    )(tableT)


def _gather_kernel(hist, d, idxT_hbm, tableH_hbm, out5_hbm, idxh_v,
                   par_v, rows_v, rowsT_v, gsems, ssems):
    wid = lax.axis_index("s") * NUM_CORES + lax.axis_index("c")

    pltpu.sync_copy(idxT_hbm.at[:, pl.ds(wid * CHUNK, CHUNK)], idxh_v)

    sh = TBLK.bit_length() - 1

    def prep_row(h, _):
        for bb in range(8):
            v = idxh_v[h, pl.ds(bb * 16, 16)]
            par_v[h, pl.ds(bb * 16, 16)] = ((v >> (sh - 1)) & 1) << 6
            idxh_v[h, pl.ds(bb * 16, 16)] = (
                ((v >> sh) << (sh - 1)) | (v & (TBLK // 2 - 1))
            )
        return 0

    lax.fori_loop(0, hist, prep_row, 0)

    def start_gather(h, buf):
        pltpu.async_copy(
            tableH_hbm.at[idxh_v.at[h]],
            rows_v.at[buf],
            gsems.at[buf],
        )

    def wait_gather(h, buf):
        pltpu.make_async_copy(
            tableH_hbm.at[idxh_v.at[h]],
            rows_v.at[buf],
            gsems.at[buf],
        ).wait()

    def start_store(h, buf):
        for jt in range(8):
            pltpu.async_copy(
                rowsT_v.at[buf, pl.ds(jt * 8, 8)],
                out5_hbm.at[h, jt, wid],
                ssems.at[buf],
            )

    def wait_store(h, buf):
        for jt in range(8):
            pltpu.make_async_copy(
                rowsT_v.at[buf, pl.ds(jt * 8, 8)],
                out5_hbm.at[h, jt, wid],
                ssems.at[buf],
            ).wait()

    lane = lax.iota(jnp.int32, 16)

    def transpose_chunk(h, buf):
        rows = rows_v.at[buf]
        rowsT = rowsT_v.at[buf]
        c_vecs = [lane + (16 * bb) for bb in range(8)]
        pars = [par_v[h, pl.ds(bb * 16, 16)] for bb in range(8)]

        def k_body(k, _):
            dg = (lane + k) & 15
            dgd0 = [dg + d0 for d0 in range(0, 64, 16)]
            vals = [
                plsc.load_gather(rows, [c_vecs[bb], pars[bb] + dgd0[di]])
                for bb in range(8) for di in range(4)
            ]
            i = 0
            for bb in range(8):
                for di in range(4):
                    plsc.store_scatter(rowsT, [dgd0[di], c_vecs[bb]], vals[i])
                    i += 1
            return 0

        lax.fori_loop(0, 16, k_body, 0)

    start_gather(0, 0)
    start_gather(1, 1)

    def body(g, _):
        for buf in range(NBUF):
            h = g * NBUF + buf
            wait_gather(h, buf)

            @pl.when(h >= NBUF)
            def _():
                wait_store(h - NBUF, buf)

            transpose_chunk(h, buf)
            start_store(h, buf)

            @pl.when(h + NBUF < hist)
            def _():
                start_gather(h + NBUF, buf)

        return 0

    lax.fori_loop(0, hist // NBUF, body, 0)

    for buf in range(NBUF):
        wait_store(hist - NBUF + buf, buf)


def kernel(item_ids, table):
    batch, hist = item_ids.shape
    n_rows, d = table.shape
    idxT = item_ids.T
    tableH = _pack_table(table.T, n_rows)

    mesh = plsc.VectorSubcoreMesh(
        core_axis_name="c",
        subcore_axis_name="s",
        num_cores=NUM_CORES,
        num_subcores=NUM_SUBCORES,
    )

    grid_kernel = pl.kernel(
        functools.partial(_gather_kernel, hist, d),
        out_type=jax.ShapeDtypeStruct((hist, 8, batch // CHUNK, 8, CHUNK),
                                      table.dtype),
        mesh=mesh,
        scratch_types=[
            pltpu.VMEM((hist, CHUNK), jnp.int32),
            pltpu.VMEM((hist, CHUNK), jnp.int32),
            pltpu.VMEM((NBUF, CHUNK, 128), table.dtype),
            pltpu.VMEM((NBUF, 64, CHUNK), table.dtype),
            pltpu.SemaphoreType.DMA((NBUF,)),
            pltpu.SemaphoreType.DMA((NBUF,)),
        ],
        compiler_params=pltpu.CompilerParams(
            use_tc_tiling_on_sc=False, needs_layout_passes=False
        ),
    )
    out5 = grid_kernel(idxT, tableH)
    return out5.transpose(2, 4, 0, 1, 3).reshape(batch, hist, d)


# speedup vs baseline: 4.8577x; 1.1241x over previous
import functools

import jax
import jax.numpy as jnp
from jax import lax
from jax.experimental import pallas as pl
from jax.experimental.pallas import tpu as pltpu
from jax.experimental.pallas import tpu_sc as plsc

NUM_CORES = 2
NUM_SUBCORES = 16
NW = NUM_CORES * NUM_SUBCORES

CHUNK = 128
NBUF = 2
TBLK = 8192


def _pack_kernel(tT_ref, out_ref):
    blk = tT_ref[...]
    eye = (lax.broadcasted_iota(jnp.int32, (64, 64), 0)
           == lax.broadcasted_iota(jnp.int32, (64, 64), 1)).astype(blk.dtype)
    t1 = lax.dot_general(blk, eye, (((0,), (0,)), ((), ())),
                         preferred_element_type=jnp.float32)
    out_ref[:, 0:64] = t1[0:TBLK // 2]
    out_ref[:, 64:128] = t1[TBLK // 2:TBLK]


def _pack_table(tableT, n_rows):
    n_blocks = (n_rows + TBLK - 1) // TBLK
    return pl.pallas_call(
        _pack_kernel,
        grid=(n_blocks,),
        in_specs=[pl.BlockSpec((64, TBLK), lambda i: (0, i))],
        out_specs=pl.BlockSpec((TBLK // 2, 128), lambda i: (i, 0)),
        out_shape=jax.ShapeDtypeStruct((n_blocks * (TBLK // 2), 128),
                                       tableT.dtype),
    )(tableT)


def _gather_kernel(hist, d, idxT_hbm, tableH_hbm, out5_hbm, idxh_v,
                   par_v, rows_v, rowsT_v, gsems, ssems):
    wid = lax.axis_index("s") * NUM_CORES + lax.axis_index("c")

    pltpu.sync_copy(idxT_hbm.at[:, pl.ds(wid * CHUNK, CHUNK)], idxh_v)

    sh = TBLK.bit_length() - 1

    def prep_row(h, _):
        for bb in range(8):
            v = idxh_v[h, pl.ds(bb * 16, 16)]
            par_v[h, pl.ds(bb * 16, 16)] = ((v >> (sh - 1)) & 1) << 6
            idxh_v[h, pl.ds(bb * 16, 16)] = (
                ((v >> sh) << (sh - 1)) | (v & (TBLK // 2 - 1))
            )
        return 0

    lax.fori_loop(0, hist, prep_row, 0)

    def start_gather(h, buf):
        pltpu.async_copy(
            tableH_hbm.at[idxh_v.at[h]],
            rows_v.at[buf],
            gsems.at[buf],
        )

    def wait_gather(h, buf):
        pltpu.make_async_copy(
            tableH_hbm.at[idxh_v.at[h]],
            rows_v.at[buf],
            gsems.at[buf],
        ).wait()

    def start_store(h, buf):
        for jt in range(8):
            pltpu.async_copy(
                rowsT_v.at[buf, pl.ds(jt * 8, 8)],
                out5_hbm.at[h, jt, wid],
                ssems.at[buf],
            )

    def wait_store(h, buf):
        for jt in range(8):
            pltpu.make_async_copy(
                rowsT_v.at[buf, pl.ds(jt * 8, 8)],
                out5_hbm.at[h, jt, wid],
                ssems.at[buf],
            ).wait()

    lane = lax.iota(jnp.int32, 16)

    def transpose_chunk(h, buf):
        rows = rows_v.at[buf]
        rowsT = rowsT_v.at[buf]
        c_vecs = [lane + (16 * bb) for bb in range(8)]
        pars = [par_v[h, pl.ds(bb * 16, 16)] for bb in range(8)]

        def k_body(k, _):
            dg = (lane + k) & 15
            dgd0 = [dg + d0 for d0 in range(0, 64, 16)]
            vals = [
                plsc.load_gather(rows, [c_vecs[bb], pars[bb] + dgd0[di]])
                for bb in range(8) for di in range(4)
            ]
            i = 0
            for bb in range(8):
                for di in range(4):
                    plsc.store_scatter(rowsT, [dgd0[di], c_vecs[bb]], vals[i])
                    i += 1
            return 0

        lax.fori_loop(0, 16, k_body, 0)

    start_gather(0, 0)
    start_gather(1, 1)

    def body(g, _):
        for buf in range(NBUF):
            h = g * NBUF + buf
            wait_gather(h, buf)

            @pl.when(h >= NBUF)
            def _():
                wait_store(h - NBUF, buf)

            transpose_chunk(h, buf)
            start_store(h, buf)

            @pl.when(h + NBUF < hist)
            def _():
                start_gather(h + NBUF, buf)

        return 0

    lax.fori_loop(0, hist // NBUF, body, 0)

    for buf in range(NBUF):
        wait_store(hist - NBUF + buf, buf)


def kernel(item_ids, table):
    batch, hist = item_ids.shape
    n_rows, d = table.shape
    idxT = item_ids.T
    tableH = _pack_table(table.T, n_rows)

    mesh = plsc.VectorSubcoreMesh(
        core_axis_name="c",
        subcore_axis_name="s",
        num_cores=NUM_CORES,
        num_subcores=NUM_SUBCORES,
    )

    grid_kernel = pl.kernel(
        functools.partial(_gather_kernel, hist, d),
        out_type=jax.ShapeDtypeStruct((hist, 8, batch // CHUNK, 8, CHUNK),
                                      table.dtype),
        mesh=mesh,
        scratch_types=[
            pltpu.VMEM((hist, CHUNK), jnp.int32),
            pltpu.VMEM((hist, CHUNK), jnp.int32),
            pltpu.VMEM((NBUF, CHUNK, 128), table.dtype),
            pltpu.VMEM((NBUF, 64, CHUNK), table.dtype),
            pltpu.SemaphoreType.DMA((NBUF,)),
            pltpu.SemaphoreType.DMA((NBUF,)),
        ],
        compiler_params=pltpu.CompilerParams(
            use_tc_tiling_on_sc=False, needs_layout_passes=False
        ),
    )
    out5 = grid_kernel(idxT, tableH)
    return out5.transpose(2, 4, 0, 1, 3).reshape(batch, hist, d)


# speedup vs baseline: 5.1765x; 1.0656x over previous
import functools

import jax
import jax.numpy as jnp
from jax import lax
from jax.experimental import pallas as pl
from jax.experimental.pallas import tpu as pltpu
from jax.experimental.pallas import tpu_sc as plsc

NUM_CORES = 2
NUM_SUBCORES = 16
NW = NUM_CORES * NUM_SUBCORES

CHUNK = 128
NBUF = 2
TBLK = 16384


def _pack_kernel(tT_ref, out_ref):
    blk = tT_ref[...]
    eye = (lax.broadcasted_iota(jnp.int32, (64, 64), 0)
           == lax.broadcasted_iota(jnp.int32, (64, 64), 1)).astype(blk.dtype)
    t1 = lax.dot_general(blk, eye, (((0,), (0,)), ((), ())),
                         preferred_element_type=jnp.float32)
    out_ref[:, 0:64] = t1[0:TBLK // 2]
    out_ref[:, 64:128] = t1[TBLK // 2:TBLK]


def _pack_table(tableT, n_rows):
    n_blocks = (n_rows + TBLK - 1) // TBLK
    return pl.pallas_call(
        _pack_kernel,
        grid=(n_blocks,),
        in_specs=[pl.BlockSpec((64, TBLK), lambda i: (0, i))],
        out_specs=pl.BlockSpec((TBLK // 2, 128), lambda i: (i, 0)),
        out_shape=jax.ShapeDtypeStruct((n_blocks * (TBLK // 2), 128),
                                       tableT.dtype),
    )(tableT)


def _gather_kernel(hist, d, idxT_hbm, tableH_hbm, out5_hbm, idxh_v,
                   par_v, rows_v, rowsT_v, gsems, ssems):
    wid = lax.axis_index("s") * NUM_CORES + lax.axis_index("c")

    pltpu.sync_copy(idxT_hbm.at[:, pl.ds(wid * CHUNK, CHUNK)], idxh_v)

    sh = TBLK.bit_length() - 1

    def prep_row(h, _):
        for bb in range(8):
            v = idxh_v[h, pl.ds(bb * 16, 16)]
            par_v[h, pl.ds(bb * 16, 16)] = ((v >> (sh - 1)) & 1) << 6
            idxh_v[h, pl.ds(bb * 16, 16)] = (
                ((v >> sh) << (sh - 1)) | (v & (TBLK // 2 - 1))
            )
        return 0

    lax.fori_loop(0, hist, prep_row, 0)

    def start_gather(h, buf):
        pltpu.async_copy(
            tableH_hbm.at[idxh_v.at[h]],
            rows_v.at[buf],
            gsems.at[buf],
        )

    def wait_gather(h, buf):
        pltpu.make_async_copy(
            tableH_hbm.at[idxh_v.at[h]],
            rows_v.at[buf],
            gsems.at[buf],
        ).wait()

    def start_store(h, buf):
        for jt in range(8):
            pltpu.async_copy(
                rowsT_v.at[buf, pl.ds(jt * 8, 8)],
                out5_hbm.at[h, jt, wid],
                ssems.at[buf],
            )

    def wait_store(h, buf):
        for jt in range(8):
            pltpu.make_async_copy(
                rowsT_v.at[buf, pl.ds(jt * 8, 8)],
                out5_hbm.at[h, jt, wid],
                ssems.at[buf],
            ).wait()

    lane = lax.iota(jnp.int32, 16)

    def transpose_chunk(h, buf):
        rows = rows_v.at[buf]
        rowsT = rowsT_v.at[buf]
        c_vecs = [lane + (16 * bb) for bb in range(8)]
        pars = [par_v[h, pl.ds(bb * 16, 16)] for bb in range(8)]

        def k_body(k, _):
            dg = (lane + k) & 15
            dgd0 = [dg + d0 for d0 in range(0, 64, 16)]
            vals = [
                plsc.load_gather(rows, [c_vecs[bb], pars[bb] + dgd0[di]])
                for bb in range(8) for di in range(4)
            ]
            i = 0
            for bb in range(8):
                for di in range(4):
                    plsc.store_scatter(rowsT, [dgd0[di], c_vecs[bb]], vals[i])
                    i += 1
            return 0

        lax.fori_loop(0, 16, k_body, 0)

    start_gather(0, 0)
    start_gather(1, 1)

    def body(g, _):
        for buf in range(NBUF):
            h = g * NBUF + buf
            wait_gather(h, buf)

            @pl.when(h >= NBUF)
            def _():
                wait_store(h - NBUF, buf)

            transpose_chunk(h, buf)
            start_store(h, buf)

            @pl.when(h + NBUF < hist)
            def _():
                start_gather(h + NBUF, buf)

        return 0

    lax.fori_loop(0, hist // NBUF, body, 0)

    for buf in range(NBUF):
        wait_store(hist - NBUF + buf, buf)


def kernel(item_ids, table):
    batch, hist = item_ids.shape
    n_rows, d = table.shape
    idxT = item_ids.T
    tableH = _pack_table(table.T, n_rows)

    mesh = plsc.VectorSubcoreMesh(
        core_axis_name="c",
        subcore_axis_name="s",
        num_cores=NUM_CORES,
        num_subcores=NUM_SUBCORES,
    )

    grid_kernel = pl.kernel(
        functools.partial(_gather_kernel, hist, d),
        out_type=jax.ShapeDtypeStruct((hist, 8, batch // CHUNK, 8, CHUNK),
                                      table.dtype),
        mesh=mesh,
        scratch_types=[
            pltpu.VMEM((hist, CHUNK), jnp.int32),
            pltpu.VMEM((hist, CHUNK), jnp.int32),
            pltpu.VMEM((NBUF, CHUNK, 128), table.dtype),
            pltpu.VMEM((NBUF, 64, CHUNK), table.dtype),
            pltpu.SemaphoreType.DMA((NBUF,)),
            pltpu.SemaphoreType.DMA((NBUF,)),
        ],
        compiler_params=pltpu.CompilerParams(
            use_tc_tiling_on_sc=False, needs_layout_passes=False
        ),
    )
    out5 = grid_kernel(idxT, tableH)
    return out5.transpose(2, 4, 0, 1, 3).reshape(batch, hist, d)


# speedup vs baseline: 5.3354x; 1.0307x over previous
import functools

import jax
import jax.numpy as jnp
from jax import lax
from jax.experimental import pallas as pl
from jax.experimental.pallas import tpu as pltpu
from jax.experimental.pallas import tpu_sc as plsc

NUM_CORES = 2
NUM_SUBCORES = 16
NW = NUM_CORES * NUM_SUBCORES

CHUNK = 128
NBUF = 2
TBLK = 32768


def _pack_kernel(tT_ref, out_ref):
    blk = tT_ref[...]
    eye = (lax.broadcasted_iota(jnp.int32, (64, 64), 0)
           == lax.broadcasted_iota(jnp.int32, (64, 64), 1)).astype(blk.dtype)
    t1 = lax.dot_general(blk, eye, (((0,), (0,)), ((), ())),
                         preferred_element_type=jnp.float32)
    out_ref[:, 0:64] = t1[0:TBLK // 2]
    out_ref[:, 64:128] = t1[TBLK // 2:TBLK]


def _pack_table(tableT, n_rows):
    n_blocks = (n_rows + TBLK - 1) // TBLK
    return pl.pallas_call(
        _pack_kernel,
        grid=(n_blocks,),
        in_specs=[pl.BlockSpec((64, TBLK), lambda i: (0, i))],
        out_specs=pl.BlockSpec((TBLK // 2, 128), lambda i: (i, 0)),
        out_shape=jax.ShapeDtypeStruct((n_blocks * (TBLK // 2), 128),
                                       tableT.dtype),
    )(tableT)


def _gather_kernel(hist, d, idxT_hbm, tableH_hbm, out5_hbm, idxh_v,
                   par_v, rows_v, rowsT_v, gsems, ssems):
    wid = lax.axis_index("s") * NUM_CORES + lax.axis_index("c")

    pltpu.sync_copy(idxT_hbm.at[:, pl.ds(wid * CHUNK, CHUNK)], idxh_v)

    sh = TBLK.bit_length() - 1

    def prep_row(h, _):
        for bb in range(8):
            v = idxh_v[h, pl.ds(bb * 16, 16)]
            par_v[h, pl.ds(bb * 16, 16)] = ((v >> (sh - 1)) & 1) << 6
            idxh_v[h, pl.ds(bb * 16, 16)] = (
                ((v >> sh) << (sh - 1)) | (v & (TBLK // 2 - 1))
            )
        return 0

    lax.fori_loop(0, hist, prep_row, 0)

    def start_gather(h, buf):
        pltpu.async_copy(
            tableH_hbm.at[idxh_v.at[h]],
            rows_v.at[buf],
            gsems.at[buf],
        )

    def wait_gather(h, buf):
        pltpu.make_async_copy(
            tableH_hbm.at[idxh_v.at[h]],
            rows_v.at[buf],
            gsems.at[buf],
        ).wait()

    def start_store(h, buf):
        for jt in range(8):
            pltpu.async_copy(
                rowsT_v.at[buf, pl.ds(jt * 8, 8)],
                out5_hbm.at[h, jt, wid],
                ssems.at[buf],
            )

    def wait_store(h, buf):
        for jt in range(8):
            pltpu.make_async_copy(
                rowsT_v.at[buf, pl.ds(jt * 8, 8)],
                out5_hbm.at[h, jt, wid],
                ssems.at[buf],
            ).wait()

    lane = lax.iota(jnp.int32, 16)

    def transpose_chunk(h, buf):
        rows = rows_v.at[buf]
        rowsT = rowsT_v.at[buf]
        c_vecs = [lane + (16 * bb) for bb in range(8)]
        pars = [par_v[h, pl.ds(bb * 16, 16)] for bb in range(8)]

        def k_body(k, _):
            dg = (lane + k) & 15
            dgd0 = [dg + d0 for d0 in range(0, 64, 16)]
            vals = [
                plsc.load_gather(rows, [c_vecs[bb], pars[bb] + dgd0[di]])
                for bb in range(8) for di in range(4)
            ]
            i = 0
            for bb in range(8):
                for di in range(4):
                    plsc.store_scatter(rowsT, [dgd0[di], c_vecs[bb]], vals[i])
                    i += 1
            return 0

        lax.fori_loop(0, 16, k_body, 0)

    start_gather(0, 0)
    start_gather(1, 1)

    def body(g, _):
        for buf in range(NBUF):
            h = g * NBUF + buf
            wait_gather(h, buf)

            @pl.when(h >= NBUF)
            def _():
                wait_store(h - NBUF, buf)

            transpose_chunk(h, buf)
            start_store(h, buf)

            @pl.when(h + NBUF < hist)
            def _():
                start_gather(h + NBUF, buf)

        return 0

    lax.fori_loop(0, hist // NBUF, body, 0)

    for buf in range(NBUF):
        wait_store(hist - NBUF + buf, buf)


def kernel(item_ids, table):
    batch, hist = item_ids.shape
    n_rows, d = table.shape
    idxT = item_ids.T
    tableH = _pack_table(table.T, n_rows)

    mesh = plsc.VectorSubcoreMesh(
        core_axis_name="c",
        subcore_axis_name="s",
        num_cores=NUM_CORES,
        num_subcores=NUM_SUBCORES,
    )

    grid_kernel = pl.kernel(
        functools.partial(_gather_kernel, hist, d),
        out_type=jax.ShapeDtypeStruct((hist, 8, batch // CHUNK, 8, CHUNK),
                                      table.dtype),
        mesh=mesh,
        scratch_types=[
            pltpu.VMEM((hist, CHUNK), jnp.int32),
            pltpu.VMEM((hist, CHUNK), jnp.int32),
            pltpu.VMEM((NBUF, CHUNK, 128), table.dtype),
            pltpu.VMEM((NBUF, 64, CHUNK), table.dtype),
            pltpu.SemaphoreType.DMA((NBUF,)),
            pltpu.SemaphoreType.DMA((NBUF,)),
        ],
        compiler_params=pltpu.CompilerParams(
            use_tc_tiling_on_sc=False, needs_layout_passes=False
        ),
    )
    out5 = grid_kernel(idxT, tableH)
    return out5.transpose(2, 4, 0, 1, 3).reshape(batch, hist, d)


# speedup vs baseline: 5.4992x; 1.0307x over previous
import functools

import jax
import jax.numpy as jnp
from jax import lax
from jax.experimental import pallas as pl
from jax.experimental.pallas import tpu as pltpu
from jax.experimental.pallas import tpu_sc as plsc

NUM_CORES = 2
NUM_SUBCORES = 16
NW = NUM_CORES * NUM_SUBCORES

CHUNK = 128
NBUF = 2
TBLK = 32768


def _pack_kernel(tT_ref, out_ref):
    blk = tT_ref[...]
    eye = (lax.broadcasted_iota(jnp.int32, (64, 64), 0)
           == lax.broadcasted_iota(jnp.int32, (64, 64), 1)).astype(blk.dtype)
    t1 = lax.dot_general(blk, eye, (((0,), (0,)), ((), ())),
                         preferred_element_type=jnp.float32)
    out_ref[:, 0:64] = t1[0:TBLK // 2]
    out_ref[:, 64:128] = t1[TBLK // 2:TBLK]


def _pack_table(tableT, n_rows):
    n_blocks = (n_rows + TBLK - 1) // TBLK
    return pl.pallas_call(
        _pack_kernel,
        grid=(n_blocks,),
        in_specs=[pl.BlockSpec((64, TBLK), lambda i: (0, i))],
        out_specs=pl.BlockSpec((TBLK // 2, 128), lambda i: (i, 0)),
        out_shape=jax.ShapeDtypeStruct((n_blocks * (TBLK // 2), 128),
                                       tableT.dtype),
    )(tableT)


def _gather_kernel(hist, d, idxT_hbm, tableH_hbm, out5_hbm, idxh_v,
                   par_v, rows_v, rowsT_v, gsems, ssems):
    wid = lax.axis_index("s") * NUM_CORES + lax.axis_index("c")

    pltpu.sync_copy(idxT_hbm.at[:, pl.ds(wid * CHUNK, CHUNK)], idxh_v)

    sh = TBLK.bit_length() - 1

    def prep_row(h, _):
        for bb in range(8):
            v = idxh_v[h, pl.ds(bb * 16, 16)]
            par_v[h, pl.ds(bb * 16, 16)] = ((v >> (sh - 1)) & 1) << 6
            idxh_v[h, pl.ds(bb * 16, 16)] = (
                ((v >> sh) << (sh - 1)) | (v & (TBLK // 2 - 1))
            )
        return 0

    lax.fori_loop(0, hist, prep_row, 0)

    def start_gather(h, buf):
        pltpu.async_copy(
            tableH_hbm.at[idxh_v.at[h]],
            rows_v.at[buf],
            gsems.at[buf],
        )

    def wait_gather(h, buf):
        pltpu.make_async_copy(
            tableH_hbm.at[idxh_v.at[h]],
            rows_v.at[buf],
            gsems.at[buf],
        ).wait()

    def start_store(h, buf):
        for jt in range(8):
            pltpu.async_copy(
                rowsT_v.at[buf, pl.ds(jt * 8, 8)],
                out5_hbm.at[h, jt, wid],
                ssems.at[buf],
            )

    def wait_store(h, buf):
        for jt in range(8):
            pltpu.make_async_copy(
                rowsT_v.at[buf, pl.ds(jt * 8, 8)],
                out5_hbm.at[h, jt, wid],
                ssems.at[buf],
            ).wait()

    lane = lax.iota(jnp.int32, 16)

    def transpose_chunk(h, buf):
        rows = rows_v.at[buf]
        rowsT = rowsT_v.at[buf]
        c_vecs = [lane + (16 * bb) for bb in range(8)]
        pars = [par_v[h, pl.ds(bb * 16, 16)] for bb in range(8)]

        def k_body(k2, _):
            for k in (2 * k2, 2 * k2 + 1):
                dg = (lane + k) & 15
                dgd0 = [dg + d0 for d0 in range(0, 64, 16)]
                vals = [
                    plsc.load_gather(rows, [c_vecs[bb], pars[bb] + dgd0[di]])
                    for bb in range(8) for di in range(4)
                ]
                i = 0
                for bb in range(8):
                    for di in range(4):
                        plsc.store_scatter(rowsT, [dgd0[di], c_vecs[bb]],
                                           vals[i])
                        i += 1
            return 0

        lax.fori_loop(0, 8, k_body, 0)

    start_gather(0, 0)
    start_gather(1, 1)

    def body(g, _):
        for buf in range(NBUF):
            h = g * NBUF + buf
            wait_gather(h, buf)

            @pl.when(h >= NBUF)
            def _():
                wait_store(h - NBUF, buf)

            transpose_chunk(h, buf)
            start_store(h, buf)

            @pl.when(h + NBUF < hist)
            def _():
                start_gather(h + NBUF, buf)

        return 0

    lax.fori_loop(0, hist // NBUF, body, 0)

    for buf in range(NBUF):
        wait_store(hist - NBUF + buf, buf)


def kernel(item_ids, table):
    batch, hist = item_ids.shape
    n_rows, d = table.shape
    idxT = item_ids.T
    tableH = _pack_table(table.T, n_rows)

    mesh = plsc.VectorSubcoreMesh(
        core_axis_name="c",
        subcore_axis_name="s",
        num_cores=NUM_CORES,
        num_subcores=NUM_SUBCORES,
    )

    grid_kernel = pl.kernel(
        functools.partial(_gather_kernel, hist, d),
        out_type=jax.ShapeDtypeStruct((hist, 8, batch // CHUNK, 8, CHUNK),
                                      table.dtype),
        mesh=mesh,
        scratch_types=[
            pltpu.VMEM((hist, CHUNK), jnp.int32),
            pltpu.VMEM((hist, CHUNK), jnp.int32),
            pltpu.VMEM((NBUF, CHUNK, 128), table.dtype),
            pltpu.VMEM((NBUF, 64, CHUNK), table.dtype),
            pltpu.SemaphoreType.DMA((NBUF,)),
            pltpu.SemaphoreType.DMA((NBUF,)),
        ],
        compiler_params=pltpu.CompilerParams(
            use_tc_tiling_on_sc=False, needs_layout_passes=False
        ),
    )
    out5 = grid_kernel(idxT, tableH)
    return out5.transpose(2, 4, 0, 1, 3).reshape(batch, hist, d)


# speedup vs baseline: 5.5191x; 1.0036x over previous
import functools

import jax
import jax.numpy as jnp
from jax import lax
from jax.experimental import pallas as pl
from jax.experimental.pallas import tpu as pltpu
from jax.experimental.pallas import tpu_sc as plsc

NUM_CORES = 2
NUM_SUBCORES = 16
NW = NUM_CORES * NUM_SUBCORES

CHUNK = 128
NBUF = 2
TBLK = 32768


def _pack_kernel(tT_ref, out_ref):
    blk = tT_ref[...]
    eye = (lax.broadcasted_iota(jnp.int32, (64, 64), 0)
           == lax.broadcasted_iota(jnp.int32, (64, 64), 1)).astype(blk.dtype)
    t1 = lax.dot_general(blk, eye, (((0,), (0,)), ((), ())),
                         preferred_element_type=jnp.float32)
    out_ref[:, 0:64] = t1[0:TBLK // 2]
    out_ref[:, 64:128] = t1[TBLK // 2:TBLK]


def _pack_table(tableT, n_rows):
    n_blocks = (n_rows + TBLK - 1) // TBLK
    return pl.pallas_call(
        _pack_kernel,
        grid=(n_blocks,),
        in_specs=[pl.BlockSpec((64, TBLK), lambda i: (0, i))],
        out_specs=pl.BlockSpec((TBLK // 2, 128), lambda i: (i, 0)),
        out_shape=jax.ShapeDtypeStruct((n_blocks * (TBLK // 2), 128),
                                       tableT.dtype),
    )(tableT)


def _gather_kernel(hist, d, idxT_hbm, tableH_hbm, out5_hbm, idxh_v,
                   par_v, rows_v, rowsT_v, gsems, ssems):
    wid = lax.axis_index("s") * NUM_CORES + lax.axis_index("c")

    pltpu.sync_copy(idxT_hbm.at[:, pl.ds(wid * CHUNK, CHUNK)], idxh_v)

    sh = TBLK.bit_length() - 1

    def prep_row(h, _):
        for bb in range(8):
            v = idxh_v[h, pl.ds(bb * 16, 16)]
            par_v[h, pl.ds(bb * 16, 16)] = ((v >> (sh - 1)) & 1) << 6
            idxh_v[h, pl.ds(bb * 16, 16)] = (
                ((v >> sh) << (sh - 1)) | (v & (TBLK // 2 - 1))
            )
        return 0

    lax.fori_loop(0, hist, prep_row, 0)

    def start_gather(h, buf):
        pltpu.async_copy(
            tableH_hbm.at[idxh_v.at[h]],
            rows_v.at[buf],
            gsems.at[buf],
        )

    def wait_gather(h, buf):
        pltpu.make_async_copy(
            tableH_hbm.at[idxh_v.at[h]],
            rows_v.at[buf],
            gsems.at[buf],
        ).wait()

    def start_store(h, buf):
        for jt in range(8):
            pltpu.async_copy(
                rowsT_v.at[buf, pl.ds(jt * 8, 8)],
                out5_hbm.at[h, jt, wid],
                ssems.at[buf],
            )

    def wait_store(h, buf):
        for jt in range(8):
            pltpu.make_async_copy(
                rowsT_v.at[buf, pl.ds(jt * 8, 8)],
                out5_hbm.at[h, jt, wid],
                ssems.at[buf],
            ).wait()

    lane = lax.iota(jnp.int32, 16)

    def transpose_chunk(h, buf):
        rows = rows_v.at[buf]
        rowsT = rowsT_v.at[buf]
        c_vecs = [lane + (16 * bb) for bb in range(8)]
        pars = [par_v[h, pl.ds(bb * 16, 16)] for bb in range(8)]

        def k_body(k2, _):
            for k in (4 * k2, 4 * k2 + 1, 4 * k2 + 2, 4 * k2 + 3):
                dg = (lane + k) & 15
                dgd0 = [dg + d0 for d0 in range(0, 64, 16)]
                vals = [
                    plsc.load_gather(rows, [c_vecs[bb], pars[bb] + dgd0[di]])
                    for bb in range(8) for di in range(4)
                ]
                i = 0
                for bb in range(8):
                    for di in range(4):
                        plsc.store_scatter(rowsT, [dgd0[di], c_vecs[bb]],
                                           vals[i])
                        i += 1
            return 0

        lax.fori_loop(0, 4, k_body, 0)

    start_gather(0, 0)
    start_gather(1, 1)

    def body(g, _):
        for buf in range(NBUF):
            h = g * NBUF + buf
            wait_gather(h, buf)

            @pl.when(h >= NBUF)
            def _():
                wait_store(h - NBUF, buf)

            transpose_chunk(h, buf)
            start_store(h, buf)

            @pl.when(h + NBUF < hist)
            def _():
                start_gather(h + NBUF, buf)

        return 0

    lax.fori_loop(0, hist // NBUF, body, 0)

    for buf in range(NBUF):
        wait_store(hist - NBUF + buf, buf)


def kernel(item_ids, table):
    batch, hist = item_ids.shape
    n_rows, d = table.shape
    idxT = item_ids.T
    tableH = _pack_table(table.T, n_rows)

    mesh = plsc.VectorSubcoreMesh(
        core_axis_name="c",
        subcore_axis_name="s",
        num_cores=NUM_CORES,
        num_subcores=NUM_SUBCORES,
    )

    grid_kernel = pl.kernel(
        functools.partial(_gather_kernel, hist, d),
        out_type=jax.ShapeDtypeStruct((hist, 8, batch // CHUNK, 8, CHUNK),
                                      table.dtype),
        mesh=mesh,
        scratch_types=[
            pltpu.VMEM((hist, CHUNK), jnp.int32),
            pltpu.VMEM((hist, CHUNK), jnp.int32),
            pltpu.VMEM((NBUF, CHUNK, 128), table.dtype),
            pltpu.VMEM((NBUF, 64, CHUNK), table.dtype),
            pltpu.SemaphoreType.DMA((NBUF,)),
            pltpu.SemaphoreType.DMA((NBUF,)),
        ],
        compiler_params=pltpu.CompilerParams(
            use_tc_tiling_on_sc=False, needs_layout_passes=False
        ),
    )
    out5 = grid_kernel(idxT, tableH)
    return out5.transpose(2, 4, 0, 1, 3).reshape(batch, hist, d)
